# baseline (speedup 1.0000x reference)
import functools

import jax
import jax.numpy as jnp
from jax import lax
from jax.experimental import pallas as pl
from jax.experimental.pallas import tpu as pltpu
from jax.experimental.pallas import tpu_sc as plsc

N = 50000
NPAD = 50176
E = 800000
EB = 256
KB = 98
SLABS = 32
SUB = 14
NB2 = 2
EPAD = SLABS * KB * EB
DUMP_ROW = NPAD - 8
NC, NS = 2, 16
ROWS_T = NPAD // NS
RB = 784

_mesh = plsc.VectorSubcoreMesh(core_axis_name="c", subcore_axis_name="s")


@functools.partial(
    pl.kernel,
    out_type=jax.ShapeDtypeStruct((NPAD, 128), jnp.float32),
    mesh=_mesh,
    scratch_types=[
        pltpu.VMEM((KB, EB), jnp.int32),
        pltpu.VMEM((KB, EB), jnp.int32),
        [pltpu.VMEM((EB, 8), jnp.float32)] * 2,
        pltpu.VMEM_SHARED((NPAD, 8), jnp.float32),
        [pltpu.SemaphoreType.DMA] * 2,
        [pltpu.SemaphoreType.DMA] * 2,
    ],
    compiler_params=pltpu.CompilerParams(use_tc_tiling_on_sc=False),
)
def _seg1(xp_hbm, srcs_hbm, dsts_hbm, zeros_hbm, out_hbm,
          src_v, dst_v, rows, acc, gsems, ssems):
    c = lax.axis_index("c")
    s = lax.axis_index("s")
    wid = c * NS + s
    t0 = s * ROWS_T
    pltpu.sync_copy(zeros_hbm.at[pl.ds(t0, ROWS_T)], acc.at[pl.ds(t0, ROWS_T)])
    plsc.subcore_barrier()
    pltpu.sync_copy(srcs_hbm.at[wid], src_v)
    pltpu.sync_copy(dsts_hbm.at[wid], dst_v)
    for b in range(2):
        pltpu.async_copy(xp_hbm.at[src_v.at[b]], rows[b], gsems[b])

    def grp(g, carry):
        for b in range(2):
            j = g * 2 + b
            pltpu.make_async_copy(xp_hbm.at[src_v.at[j]], rows[b],
                                  gsems[b]).wait()
            pltpu.async_copy(rows[b], acc.at[dst_v.at[j]], ssems[b], add=True)

            @pl.when(g < KB // 2 - 1)
            def _():
                pltpu.make_async_copy(rows[b], acc.at[dst_v.at[j]],
                                      ssems[b]).wait()
                pltpu.async_copy(xp_hbm.at[src_v.at[j + 2]], rows[b], gsems[b])
        return carry

    lax.fori_loop(0, KB // 2, grp, 0)
    for b in range(2):
        pltpu.make_async_copy(rows[b], acc.at[dst_v.at[KB - 2 + b]],
                              ssems[b]).wait()
    plsc.subcore_barrier()
    pltpu.sync_copy(acc.at[pl.ds(t0, ROWS_T)],
                    out_hbm.at[pl.ds(t0, ROWS_T), pl.ds(c * 8, 8)])


@functools.partial(
    pl.kernel,
    out_type=jax.ShapeDtypeStruct((NPAD, 128), jnp.float32),
    mesh=_mesh,
    scratch_types=[
        pltpu.VMEM((SUB, EB), jnp.int32),
        pltpu.VMEM((SUB, EB), jnp.int32),
        [pltpu.VMEM((EB, 32), jnp.float32)] * 2,
        pltpu.VMEM_SHARED((NPAD, 32), jnp.float32),
        [pltpu.SemaphoreType.DMA] * 2,
        [pltpu.SemaphoreType.DMA] * 2,
    ],
    compiler_params=pltpu.CompilerParams(use_tc_tiling_on_sc=False),
)
def _seg2(g4_hbm, srcs4_hbm, dsts_hbm, zeros_hbm, out_hbm,
          src_v, dst_v, rows, acc, gsems, ssems):
    c = lax.axis_index("c")
    s = lax.axis_index("s")
    t0 = s * ROWS_T

    for cc in range(2):
        chunk = c * 2 + cc
        pltpu.sync_copy(zeros_hbm.at[pl.ds(t0, ROWS_T)],
                        acc.at[pl.ds(t0, ROWS_T)])
        plsc.subcore_barrier()
        for m in range(2):
            slab = s * 2 + m
            for q in range(KB // SUB):
                pltpu.sync_copy(
                    srcs4_hbm.at[chunk].at[slab].at[pl.ds(q * SUB, SUB)],
                    src_v)
                pltpu.sync_copy(
                    dsts_hbm.at[slab].at[pl.ds(q * SUB, SUB)], dst_v)
                for b in range(2):
                    pltpu.async_copy(g4_hbm.at[src_v.at[b]], rows[b], gsems[b])

                def grp(g, carry):
                    for b in range(2):
                        j = g * 2 + b
                        pltpu.make_async_copy(g4_hbm.at[src_v.at[j]], rows[b],
                                              gsems[b]).wait()
                        pltpu.async_copy(rows[b], acc.at[dst_v.at[j]],
                                         ssems[b], add=True)

                        @pl.when(g < SUB // 2 - 1)
                        def _():
                            pltpu.make_async_copy(rows[b], acc.at[dst_v.at[j]],
                                                  ssems[b]).wait()
                            pltpu.async_copy(g4_hbm.at[src_v.at[j + 2]],
                                             rows[b], gsems[b])
                    return carry

                lax.fori_loop(0, SUB // 2, grp, 0)
                for b in range(2):
                    pltpu.make_async_copy(rows[b],
                                          acc.at[dst_v.at[SUB - 2 + b]],
                                          ssems[b]).wait()
        plsc.subcore_barrier()
        pltpu.sync_copy(acc.at[pl.ds(t0, ROWS_T)],
                        out_hbm.at[pl.ds(t0, ROWS_T), pl.ds(chunk * 32, 32)])
        plsc.subcore_barrier()


def _enc_body(a_ref, x_ref, w16_ref, wr1_ref, b1_ref, w2_ref, wr2_ref,
              g_ref, r_ref):
    h1 = jnp.maximum(
        jnp.dot(a_ref[:, 0:16], w16_ref[...],
                preferred_element_type=jnp.float32)
        + b1_ref[...]
        + jnp.dot(x_ref[...], wr1_ref[...], preferred_element_type=jnp.float32),
        0.0)
    g_ref[...] = jnp.dot(h1, w2_ref[...], preferred_element_type=jnp.float32)
    r_ref[...] = jnp.dot(h1, wr2_ref[...], preferred_element_type=jnp.float32)


def _dec_body(a_ref, r_ref, b2_ref, wd_ref, bd_ref, wo_ref, bo_ref, o_ref):
    h2 = jnp.maximum(a_ref[...] + b2_ref[...] + r_ref[...], 0.0)
    z = jnp.dot(h2, wd_ref[...], preferred_element_type=jnp.float32) + bd_ref[...]
    hh = jnp.where(z > 0, z, jnp.exp(jnp.minimum(z, 0.0)) - 1.0)
    o_ref[...] = jnp.dot(hh, wo_ref[...],
                         preferred_element_type=jnp.float32) + bo_ref[...]


def _full(shape):
    return pl.BlockSpec(shape, lambda i: tuple(0 for _ in shape))


_encoder = pl.pallas_call(
    _enc_body,
    grid=(NPAD // RB,),
    in_specs=[
        pl.BlockSpec((RB, 128), lambda i: (i, 0)),
        pl.BlockSpec((RB, 6), lambda i: (i, 0)),
        _full((16, 128)), _full((6, 128)), _full((1, 128)),
        _full((128, 128)), _full((128, 128)),
    ],
    out_specs=[
        pl.BlockSpec((RB, 128), lambda i: (i, 0)),
        pl.BlockSpec((RB, 128), lambda i: (i, 0)),
    ],
    out_shape=[
        jax.ShapeDtypeStruct((NPAD, 128), jnp.float32),
        jax.ShapeDtypeStruct((NPAD, 128), jnp.float32),
    ],
)

_decoder = pl.pallas_call(
    _dec_body,
    grid=(NPAD // RB,),
    in_specs=[
        pl.BlockSpec((RB, 128), lambda i: (i, 0)),
        pl.BlockSpec((RB, 128), lambda i: (i, 0)),
        _full((1, 128)), _full((128, 192)), _full((1, 192)),
        _full((192, 8)), _full((1, 8)),
    ],
    out_specs=pl.BlockSpec((RB, 8), lambda i: (i, 0)),
    out_shape=jax.ShapeDtypeStruct((NPAD, 8), jnp.float32),
)


def kernel(x, edge_index,
           W_rel1, b_rel1, W_root1,
           W_rel2, b_rel2, W_root2,
           W_lin, b_lin,
           W_d1_0, b_d1_0, W_d1_o, b_d1_o,
           W_d2_0, b_d2_0, W_d2_o, b_d2_o,
           W_d3_0, b_d3_0, W_d3_o, b_d3_o):
    x0 = x[0]
    src = edge_index[0, 0]
    dst = edge_index[0, 1]

    xp = jnp.zeros((NPAD, 8), jnp.float32).at[:N, :6].set(x0)
    src_p = jnp.concatenate([src, jnp.zeros((EPAD - E,), jnp.int32)])
    srcs = src_p.reshape(SLABS, KB, EB)
    srcs4 = (src_p[None, :] * 4
             + jnp.arange(4, dtype=jnp.int32)[:, None]).reshape(
                 4, SLABS, KB, EB)
    dsts = jnp.concatenate(
        [dst, jnp.full((EPAD - E,), DUMP_ROW, jnp.int32)]).reshape(
            SLABS, KB, EB)
    zeros8 = jnp.zeros((NPAD, 8), jnp.float32)
    zeros32 = jnp.zeros((NPAD, 32), jnp.float32)

    w16 = jnp.zeros((16, 128), jnp.float32).at[0:6].set(W_rel1).at[8:14].set(
        W_rel1)
    b1 = b_rel1.reshape(1, 128)
    b2 = b_rel2.reshape(1, 128)
    wd = jnp.concatenate(
        [W_lin[:, 0:64] @ W_d1_0, W_lin[:, 64:128] @ W_d2_0,
         W_lin[:, 128:192] @ W_d3_0], axis=1)
    bd = jnp.concatenate(
        [b_lin[0:64] @ W_d1_0 + b_d1_0, b_lin[64:128] @ W_d2_0 + b_d2_0,
         b_lin[128:192] @ W_d3_0 + b_d3_0]).reshape(1, 192)
    wo = jnp.zeros((192, 8), jnp.float32)
    wo = wo.at[0:64, 0].set(W_d1_o[:, 0])
    wo = wo.at[64:128, 1].set(W_d2_o[:, 0])
    wo = wo.at[128:192, 2].set(W_d3_o[:, 0])
    bo = jnp.zeros((1, 8), jnp.float32)
    bo = bo.at[0, 0].set(b_d1_o[0]).at[0, 1].set(b_d2_o[0]).at[0, 2].set(b_d3_o[0])

    agg1 = _seg1(xp, srcs, dsts, zeros8)
    g, r1 = _encoder(agg1, x0, w16, W_root1, b1, W_rel2, W_root2)
    agg2 = _seg2(g.reshape(4 * NPAD, 32), srcs4, dsts, zeros32)
    out = _decoder(agg2, r1, b2, wd, bd, wo, bo)
    return out[:N, :3].reshape(N, 3, 1)


# speedup vs baseline: 1.1318x; 1.1318x over previous
import functools

import jax
import jax.numpy as jnp
from jax import lax
from jax.experimental import pallas as pl
from jax.experimental.pallas import tpu as pltpu
from jax.experimental.pallas import tpu_sc as plsc

N = 50000
NPAD = 50176
E = 800000
EB = 128
KB = 196
SLABS = 32
SUB = 28
EPAD = SLABS * KB * EB
DUMP_ROW = NPAD - 8
NC, NS = 2, 16
ROWS_T = NPAD // NS
RB = 784

_mesh = plsc.VectorSubcoreMesh(core_axis_name="c", subcore_axis_name="s")


@functools.partial(
    pl.kernel,
    out_type=jax.ShapeDtypeStruct((NPAD, 128), jnp.float32),
    mesh=_mesh,
    scratch_types=[
        pltpu.VMEM((KB, EB), jnp.int32),
        pltpu.VMEM((KB, EB), jnp.int32),
        [pltpu.VMEM((EB, 8), jnp.float32)] * 4,
        pltpu.VMEM_SHARED((NPAD, 8), jnp.float32),
        [pltpu.SemaphoreType.DMA] * 4,
        [pltpu.SemaphoreType.DMA] * 4,
    ],
    compiler_params=pltpu.CompilerParams(use_tc_tiling_on_sc=False),
)
def _seg1(xp_hbm, srcs_hbm, dsts_hbm, zeros_hbm, out_hbm,
          src_v, dst_v, rows, acc, gsems, ssems):
    c = lax.axis_index("c")
    s = lax.axis_index("s")
    wid = c * NS + s
    t0 = s * ROWS_T
    pltpu.sync_copy(zeros_hbm.at[pl.ds(t0, ROWS_T)], acc.at[pl.ds(t0, ROWS_T)])
    plsc.subcore_barrier()
    pltpu.sync_copy(srcs_hbm.at[wid], src_v)
    pltpu.sync_copy(dsts_hbm.at[wid], dst_v)
    for b in range(4):
        pltpu.async_copy(xp_hbm.at[src_v.at[b]], rows[b], gsems[b])

    def grp(g, carry):
        for b in range(4):
            j = g * 4 + b
            pltpu.make_async_copy(xp_hbm.at[src_v.at[j]], rows[b],
                                  gsems[b]).wait()
            pltpu.async_copy(rows[b], acc.at[dst_v.at[j]], ssems[b], add=True)

            @pl.when(g < KB // 4 - 1)
            def _():
                pltpu.make_async_copy(rows[b], acc.at[dst_v.at[j]],
                                      ssems[b]).wait()
                pltpu.async_copy(xp_hbm.at[src_v.at[j + 4]], rows[b], gsems[b])
        return carry

    lax.fori_loop(0, KB // 4, grp, 0)
    for b in range(4):
        pltpu.make_async_copy(rows[b], acc.at[dst_v.at[KB - 4 + b]],
                              ssems[b]).wait()
    plsc.subcore_barrier()
    pltpu.sync_copy(acc.at[pl.ds(t0, ROWS_T)],
                    out_hbm.at[pl.ds(t0, ROWS_T), pl.ds(c * 8, 8)])


@functools.partial(
    pl.kernel,
    out_type=jax.ShapeDtypeStruct((NPAD, 128), jnp.bfloat16),
    mesh=_mesh,
    scratch_types=[
        pltpu.VMEM((SUB, EB), jnp.int32),
        pltpu.VMEM((SUB, EB), jnp.int32),
        [pltpu.VMEM((EB, 64), jnp.bfloat16)] * 4,
        pltpu.VMEM_SHARED((NPAD, 64), jnp.bfloat16),
        [pltpu.SemaphoreType.DMA] * 4,
        [pltpu.SemaphoreType.DMA] * 4,
    ],
    compiler_params=pltpu.CompilerParams(use_tc_tiling_on_sc=False),
)
def _seg2(g2_hbm, srcs2_hbm, dsts_hbm, zeros_hbm, out_hbm,
          src_v, dst_v, rows, acc, gsems, ssems):
    c = lax.axis_index("c")
    s = lax.axis_index("s")
    t0 = s * ROWS_T

    pltpu.sync_copy(zeros_hbm.at[pl.ds(t0, ROWS_T)],
                    acc.at[pl.ds(t0, ROWS_T)])
    plsc.subcore_barrier()
    for m in range(2):
        slab = s * 2 + m
        for q in range(KB // SUB):
            pltpu.sync_copy(
                srcs2_hbm.at[c].at[slab].at[pl.ds(q * SUB, SUB)],
                src_v)
            pltpu.sync_copy(
                dsts_hbm.at[slab].at[pl.ds(q * SUB, SUB)], dst_v)
            for b in range(4):
                pltpu.async_copy(g2_hbm.at[src_v.at[b]], rows[b], gsems[b])

            def grp(g, carry):
                for b in range(4):
                    j = g * 4 + b
                    pltpu.make_async_copy(g2_hbm.at[src_v.at[j]], rows[b],
                                          gsems[b]).wait()
                    pltpu.async_copy(rows[b], acc.at[dst_v.at[j]],
                                     ssems[b], add=True)

                    @pl.when(g < SUB // 4 - 1)
                    def _():
                        pltpu.make_async_copy(rows[b], acc.at[dst_v.at[j]],
                                              ssems[b]).wait()
                        pltpu.async_copy(g2_hbm.at[src_v.at[j + 4]],
                                         rows[b], gsems[b])
                return carry

            lax.fori_loop(0, SUB // 4, grp, 0)
            for b in range(4):
                pltpu.make_async_copy(rows[b],
                                      acc.at[dst_v.at[SUB - 4 + b]],
                                      ssems[b]).wait()
    plsc.subcore_barrier()
    pltpu.sync_copy(acc.at[pl.ds(t0, ROWS_T)],
                    out_hbm.at[pl.ds(t0, ROWS_T), pl.ds(c * 64, 64)])


def _enc_body(a_ref, x_ref, w16_ref, wr1_ref, b1_ref, w2_ref, wr2_ref,
              g_ref, r_ref):
    h1 = jnp.maximum(
        jnp.dot(a_ref[:, 0:16], w16_ref[...],
                preferred_element_type=jnp.float32)
        + b1_ref[...]
        + jnp.dot(x_ref[...], wr1_ref[...], preferred_element_type=jnp.float32),
        0.0)
    g_ref[...] = jnp.dot(h1, w2_ref[...], preferred_element_type=jnp.float32)
    r_ref[...] = jnp.dot(h1, wr2_ref[...], preferred_element_type=jnp.float32)


def _dec_body(a_ref, r_ref, b2_ref, wd_ref, bd_ref, wo_ref, bo_ref, o_ref):
    h2 = jnp.maximum(a_ref[...] + b2_ref[...] + r_ref[...], 0.0)
    z = jnp.dot(h2, wd_ref[...], preferred_element_type=jnp.float32) + bd_ref[...]
    hh = jnp.where(z > 0, z, jnp.exp(jnp.minimum(z, 0.0)) - 1.0)
    o_ref[...] = jnp.dot(hh, wo_ref[...],
                         preferred_element_type=jnp.float32) + bo_ref[...]


def _full(shape):
    return pl.BlockSpec(shape, lambda i: tuple(0 for _ in shape))


_encoder = pl.pallas_call(
    _enc_body,
    grid=(NPAD // RB,),
    in_specs=[
        pl.BlockSpec((RB, 128), lambda i: (i, 0)),
        pl.BlockSpec((RB, 6), lambda i: (i, 0)),
        _full((16, 128)), _full((6, 128)), _full((1, 128)),
        _full((128, 128)), _full((128, 128)),
    ],
    out_specs=[
        pl.BlockSpec((RB, 128), lambda i: (i, 0)),
        pl.BlockSpec((RB, 128), lambda i: (i, 0)),
    ],
    out_shape=[
        jax.ShapeDtypeStruct((NPAD, 128), jnp.float32),
        jax.ShapeDtypeStruct((NPAD, 128), jnp.float32),
    ],
)

_decoder = pl.pallas_call(
    _dec_body,
    grid=(NPAD // RB,),
    in_specs=[
        pl.BlockSpec((RB, 128), lambda i: (i, 0)),
        pl.BlockSpec((RB, 128), lambda i: (i, 0)),
        _full((1, 128)), _full((128, 192)), _full((1, 192)),
        _full((192, 8)), _full((1, 8)),
    ],
    out_specs=pl.BlockSpec((RB, 8), lambda i: (i, 0)),
    out_shape=jax.ShapeDtypeStruct((NPAD, 8), jnp.float32),
)


def kernel(x, edge_index,
           W_rel1, b_rel1, W_root1,
           W_rel2, b_rel2, W_root2,
           W_lin, b_lin,
           W_d1_0, b_d1_0, W_d1_o, b_d1_o,
           W_d2_0, b_d2_0, W_d2_o, b_d2_o,
           W_d3_0, b_d3_0, W_d3_o, b_d3_o):
    x0 = x[0]
    src = edge_index[0, 0]
    dst = edge_index[0, 1]

    xp = jnp.zeros((NPAD, 8), jnp.float32).at[:N, :6].set(x0)
    src_p = jnp.concatenate([src, jnp.zeros((EPAD - E,), jnp.int32)])
    srcs = src_p.reshape(SLABS, KB, EB)
    srcs2 = (src_p[None, :] * 2
             + jnp.arange(2, dtype=jnp.int32)[:, None]).reshape(
                 2, SLABS, KB, EB)
    dsts = jnp.concatenate(
        [dst, jnp.full((EPAD - E,), DUMP_ROW, jnp.int32)]).reshape(
            SLABS, KB, EB)
    zeros8 = jnp.zeros((NPAD, 8), jnp.float32)
    zeros64 = jnp.zeros((NPAD, 64), jnp.bfloat16)

    w16 = jnp.zeros((16, 128), jnp.float32).at[0:6].set(W_rel1).at[8:14].set(
        W_rel1)
    b1 = b_rel1.reshape(1, 128)
    b2 = b_rel2.reshape(1, 128)
    wd = jnp.concatenate(
        [W_lin[:, 0:64] @ W_d1_0, W_lin[:, 64:128] @ W_d2_0,
         W_lin[:, 128:192] @ W_d3_0], axis=1)
    bd = jnp.concatenate(
        [b_lin[0:64] @ W_d1_0 + b_d1_0, b_lin[64:128] @ W_d2_0 + b_d2_0,
         b_lin[128:192] @ W_d3_0 + b_d3_0]).reshape(1, 192)
    wo = jnp.zeros((192, 8), jnp.float32)
    wo = wo.at[0:64, 0].set(W_d1_o[:, 0])
    wo = wo.at[64:128, 1].set(W_d2_o[:, 0])
    wo = wo.at[128:192, 2].set(W_d3_o[:, 0])
    bo = jnp.zeros((1, 8), jnp.float32)
    bo = bo.at[0, 0].set(b_d1_o[0]).at[0, 1].set(b_d2_o[0]).at[0, 2].set(b_d3_o[0])

    agg1 = _seg1(xp, srcs, dsts, zeros8)
    g, r1 = _encoder(agg1, x0, w16, W_root1, b1, W_rel2, W_root2)
    g2 = g.astype(jnp.bfloat16).reshape(2 * NPAD, 64)
    agg2 = _seg2(g2, srcs2, dsts, zeros64).astype(jnp.float32)
    out = _decoder(agg2, r1, b2, wd, bd, wo, bo)
    return out[:N, :3].reshape(N, 3, 1)


# speedup vs baseline: 1.1783x; 1.0410x over previous
import functools

import jax
import jax.numpy as jnp
from jax import lax
from jax.experimental import pallas as pl
from jax.experimental.pallas import tpu as pltpu
from jax.experimental.pallas import tpu_sc as plsc

N = 50000
NPAD = 50176
E = 800000
EB = 128
KB = 196
SLABS = 32
SUB = 28
EPAD = SLABS * KB * EB
DUMP_ROW = NPAD - 8
NC, NS = 2, 16
ROWS_T = NPAD // NS
RB = 784

_mesh = plsc.VectorSubcoreMesh(core_axis_name="c", subcore_axis_name="s")


@functools.partial(
    pl.kernel,
    out_type=jax.ShapeDtypeStruct((NPAD, 128), jnp.float32),
    mesh=_mesh,
    scratch_types=[
        pltpu.VMEM((KB, EB), jnp.int32),
        pltpu.VMEM((KB, EB), jnp.int32),
        [pltpu.VMEM((EB, 8), jnp.float32)] * 4,
        pltpu.VMEM_SHARED((NPAD, 8), jnp.float32),
        [pltpu.SemaphoreType.DMA] * 4,
        [pltpu.SemaphoreType.DMA] * 4,
    ],
    compiler_params=pltpu.CompilerParams(use_tc_tiling_on_sc=False),
)
def _seg1(xp_hbm, srcs_hbm, dsts_hbm, zeros_hbm, out_hbm,
          src_v, dst_v, rows, acc, gsems, ssems):
    c = lax.axis_index("c")
    s = lax.axis_index("s")
    wid = c * NS + s
    t0 = s * ROWS_T
    pltpu.sync_copy(zeros_hbm.at[pl.ds(t0, ROWS_T)], acc.at[pl.ds(t0, ROWS_T)])
    plsc.subcore_barrier()
    pltpu.sync_copy(srcs_hbm.at[wid], src_v)
    pltpu.sync_copy(dsts_hbm.at[wid], dst_v)
    for b in range(4):
        pltpu.async_copy(xp_hbm.at[src_v.at[b]], rows[b], gsems[b])

    def grp(g, carry):
        for b in range(4):
            j = g * 4 + b
            pltpu.make_async_copy(xp_hbm.at[src_v.at[j]], rows[b],
                                  gsems[b]).wait()
            pltpu.async_copy(rows[b], acc.at[dst_v.at[j]], ssems[b], add=True)

            @pl.when(g < KB // 4 - 1)
            def _():
                pltpu.make_async_copy(rows[b], acc.at[dst_v.at[j]],
                                      ssems[b]).wait()
                pltpu.async_copy(xp_hbm.at[src_v.at[j + 4]], rows[b], gsems[b])
        return carry

    lax.fori_loop(0, KB // 4, grp, 0)
    for b in range(4):
        pltpu.make_async_copy(rows[b], acc.at[dst_v.at[KB - 4 + b]],
                              ssems[b]).wait()
    plsc.subcore_barrier()
    pltpu.sync_copy(acc.at[pl.ds(t0, ROWS_T)],
                    out_hbm.at[pl.ds(t0, ROWS_T), pl.ds(c * 8, 8)])


@functools.partial(
    pl.kernel,
    out_type=jax.ShapeDtypeStruct((NPAD, 128), jnp.bfloat16),
    mesh=_mesh,
    scratch_types=[
        pltpu.VMEM((SUB, EB), jnp.int32),
        pltpu.VMEM((SUB, EB), jnp.int32),
        [pltpu.VMEM((EB, 64), jnp.bfloat16)] * 4,
        pltpu.VMEM_SHARED((NPAD, 64), jnp.bfloat16),
        [pltpu.SemaphoreType.DMA] * 4,
        [pltpu.SemaphoreType.DMA] * 4,
    ],
    compiler_params=pltpu.CompilerParams(use_tc_tiling_on_sc=False),
)
def _seg2(g2_hbm, srcs2_hbm, dsts_hbm, zeros_hbm, out_hbm,
          src_v, dst_v, rows, acc, gsems, ssems):
    c = lax.axis_index("c")
    s = lax.axis_index("s")
    t0 = s * ROWS_T

    pltpu.sync_copy(zeros_hbm.at[pl.ds(t0, ROWS_T)],
                    acc.at[pl.ds(t0, ROWS_T)])
    plsc.subcore_barrier()
    for m in range(2):
        slab = s * 2 + m
        for q in range(KB // SUB):
            pltpu.sync_copy(
                srcs2_hbm.at[c].at[slab].at[pl.ds(q * SUB, SUB)],
                src_v)
            pltpu.sync_copy(
                dsts_hbm.at[slab].at[pl.ds(q * SUB, SUB)], dst_v)
            for b in range(4):
                pltpu.async_copy(g2_hbm.at[src_v.at[b]], rows[b], gsems[b])

            def grp(g, carry):
                for b in range(4):
                    j = g * 4 + b
                    pltpu.make_async_copy(g2_hbm.at[src_v.at[j]], rows[b],
                                          gsems[b]).wait()
                    pltpu.async_copy(rows[b], acc.at[dst_v.at[j]],
                                     ssems[b], add=True)

                    @pl.when(g < SUB // 4 - 1)
                    def _():
                        pltpu.make_async_copy(rows[b], acc.at[dst_v.at[j]],
                                              ssems[b]).wait()
                        pltpu.async_copy(g2_hbm.at[src_v.at[j + 4]],
                                         rows[b], gsems[b])
                return carry

            lax.fori_loop(0, SUB // 4, grp, 0)
            for b in range(4):
                pltpu.make_async_copy(rows[b],
                                      acc.at[dst_v.at[SUB - 4 + b]],
                                      ssems[b]).wait()
    plsc.subcore_barrier()
    pltpu.sync_copy(acc.at[pl.ds(t0, ROWS_T)],
                    out_hbm.at[pl.ds(t0, ROWS_T), pl.ds(c * 64, 64)])


def _enc_body(a_ref, x_ref, w16_ref, wr1_ref, b1_ref, w2_ref, wr2_ref,
              g_ref, r_ref):
    h1 = jnp.maximum(
        jnp.dot(a_ref[:, 0:16], w16_ref[...],
                preferred_element_type=jnp.float32)
        + b1_ref[...]
        + jnp.dot(x_ref[...], wr1_ref[...], preferred_element_type=jnp.float32),
        0.0)
    g_ref[...] = jnp.dot(h1, w2_ref[...],
                         preferred_element_type=jnp.float32).astype(jnp.bfloat16)
    r_ref[...] = jnp.dot(h1, wr2_ref[...], preferred_element_type=jnp.float32)


def _dec_body(a_ref, r_ref, b2_ref, wd_ref, bd_ref, wo_ref, bo_ref, o_ref):
    h2 = jnp.maximum(a_ref[...].astype(jnp.float32) + b2_ref[...] + r_ref[...],
                     0.0)
    z = jnp.dot(h2, wd_ref[...], preferred_element_type=jnp.float32) + bd_ref[...]
    hh = jnp.where(z > 0, z, jnp.exp(jnp.minimum(z, 0.0)) - 1.0)
    o_ref[...] = jnp.dot(hh, wo_ref[...],
                         preferred_element_type=jnp.float32) + bo_ref[...]


def _full(shape):
    return pl.BlockSpec(shape, lambda i: tuple(0 for _ in shape))


_encoder = pl.pallas_call(
    _enc_body,
    grid=(NPAD // RB,),
    in_specs=[
        pl.BlockSpec((RB, 128), lambda i: (i, 0)),
        pl.BlockSpec((RB, 6), lambda i: (i, 0)),
        _full((16, 128)), _full((6, 128)), _full((1, 128)),
        _full((128, 128)), _full((128, 128)),
    ],
    out_specs=[
        pl.BlockSpec((RB, 128), lambda i: (i, 0)),
        pl.BlockSpec((RB, 128), lambda i: (i, 0)),
    ],
    out_shape=[
        jax.ShapeDtypeStruct((NPAD, 128), jnp.bfloat16),
        jax.ShapeDtypeStruct((NPAD, 128), jnp.float32),
    ],
)

_decoder = pl.pallas_call(
    _dec_body,
    grid=(NPAD // RB,),
    in_specs=[
        pl.BlockSpec((RB, 128), lambda i: (i, 0)),
        pl.BlockSpec((RB, 128), lambda i: (i, 0)),
        _full((1, 128)), _full((128, 192)), _full((1, 192)),
        _full((192, 8)), _full((1, 8)),
    ],
    out_specs=pl.BlockSpec((RB, 8), lambda i: (i, 0)),
    out_shape=jax.ShapeDtypeStruct((NPAD, 8), jnp.float32),
)


def kernel(x, edge_index,
           W_rel1, b_rel1, W_root1,
           W_rel2, b_rel2, W_root2,
           W_lin, b_lin,
           W_d1_0, b_d1_0, W_d1_o, b_d1_o,
           W_d2_0, b_d2_0, W_d2_o, b_d2_o,
           W_d3_0, b_d3_0, W_d3_o, b_d3_o):
    x0 = x[0]
    src = edge_index[0, 0]
    dst = edge_index[0, 1]

    xp = jnp.zeros((NPAD, 8), jnp.float32).at[:N, :6].set(x0)
    src_p = jnp.concatenate([src, jnp.zeros((EPAD - E,), jnp.int32)])
    srcs = src_p.reshape(SLABS, KB, EB)
    srcs2 = (src_p[None, :] * 2
             + jnp.arange(2, dtype=jnp.int32)[:, None]).reshape(
                 2, SLABS, KB, EB)
    dsts = jnp.concatenate(
        [dst, jnp.full((EPAD - E,), DUMP_ROW, jnp.int32)]).reshape(
            SLABS, KB, EB)
    zeros8 = jnp.zeros((NPAD, 8), jnp.float32)
    zeros64 = jnp.zeros((NPAD, 64), jnp.bfloat16)

    w16 = jnp.zeros((16, 128), jnp.float32).at[0:6].set(W_rel1).at[8:14].set(
        W_rel1)
    b1 = b_rel1.reshape(1, 128)
    b2 = b_rel2.reshape(1, 128)
    wd = jnp.concatenate(
        [W_lin[:, 0:64] @ W_d1_0, W_lin[:, 64:128] @ W_d2_0,
         W_lin[:, 128:192] @ W_d3_0], axis=1)
    bd = jnp.concatenate(
        [b_lin[0:64] @ W_d1_0 + b_d1_0, b_lin[64:128] @ W_d2_0 + b_d2_0,
         b_lin[128:192] @ W_d3_0 + b_d3_0]).reshape(1, 192)
    wo = jnp.zeros((192, 8), jnp.float32)
    wo = wo.at[0:64, 0].set(W_d1_o[:, 0])
    wo = wo.at[64:128, 1].set(W_d2_o[:, 0])
    wo = wo.at[128:192, 2].set(W_d3_o[:, 0])
    bo = jnp.zeros((1, 8), jnp.float32)
    bo = bo.at[0, 0].set(b_d1_o[0]).at[0, 1].set(b_d2_o[0]).at[0, 2].set(b_d3_o[0])

    agg1 = _seg1(xp, srcs, dsts, zeros8)
    g, r1 = _encoder(agg1, x0, w16, W_root1, b1, W_rel2, W_root2)
    g2 = g.reshape(2 * NPAD, 64)
    agg2 = _seg2(g2, srcs2, dsts, zeros64)
    out = _decoder(agg2, r1, b2, wd, bd, wo, bo)
    return out[:N, :3].reshape(N, 3, 1)
